# Initial kernel scaffold; baseline (speedup 1.0000x reference)
import jax
import jax.numpy as jnp
from jax.experimental import pallas as pl


def kernel(input_ids, id_map, table, gamma, beta):
    raise NotImplementedError("write your pallas kernel here")


# speedup vs baseline: 7.5125x; 7.5125x over previous
import functools

import jax
import jax.numpy as jnp
from jax import lax
from jax.experimental import pallas as pl
from jax.experimental.pallas import tpu as pltpu
from jax.experimental.pallas import tpu_sc as plsc

D = 120
EPS = 1e-12
NC = 2
NS = 16
NW = NC * NS


def _gather_rows_sc(table, idx, chunk):
    B = idx.shape[0]
    V, d = table.shape
    b_per_w = B // NW
    n_chunks = b_per_w // chunk
    assert b_per_w * NW == B and n_chunks * chunk == b_per_w
    mesh = plsc.VectorSubcoreMesh(core_axis_name="c", subcore_axis_name="s")

    @functools.partial(
        pl.kernel,
        out_type=jax.ShapeDtypeStruct((B, d), table.dtype),
        mesh=mesh,
        compiler_params=pltpu.CompilerParams(use_tc_tiling_on_sc=False),
        scratch_types=[
            pltpu.VMEM((b_per_w,), jnp.int32),
            pltpu.VMEM((chunk, d), table.dtype),
            pltpu.SemaphoreType.DMA,
        ],
    )
    def k(table_hbm, idx_hbm, out_hbm, idx_v, rows_v, sem):
        wid = lax.axis_index("s") * NC + lax.axis_index("c")
        base = wid * b_per_w
        pltpu.sync_copy(idx_hbm.at[pl.ds(base, b_per_w)], idx_v)

        def body(i, _):
            off = i * chunk
            pltpu.async_copy(
                table_hbm.at[idx_v.at[pl.ds(off, chunk)]], rows_v, sem
            ).wait()
            pltpu.sync_copy(rows_v, out_hbm.at[pl.ds(base + off, chunk)])
            return ()

        lax.fori_loop(0, n_chunks, body, ())

    return k(table, idx)


def _layernorm_tc(x, gamma, beta, blk):
    rows, d = x.shape

    def body(x_ref, g_ref, b_ref, o_ref):
        v = x_ref[...]
        mean = jnp.mean(v, axis=-1, keepdims=True)
        cen = v - mean
        var = jnp.mean(cen * cen, axis=-1, keepdims=True)
        o_ref[...] = cen * lax.rsqrt(var + EPS) * g_ref[...] + b_ref[...]

    return pl.pallas_call(
        body,
        grid=(rows // blk,),
        in_specs=[
            pl.BlockSpec((blk, d), lambda i: (i, 0)),
            pl.BlockSpec((1, d), lambda i: (0, 0)),
            pl.BlockSpec((1, d), lambda i: (0, 0)),
        ],
        out_specs=pl.BlockSpec((blk, d), lambda i: (i, 0)),
        out_shape=jax.ShapeDtypeStruct((rows, d), jnp.float32),
    )(x, gamma, beta)


def kernel(input_ids, id_map, table, gamma, beta):
    batch, seq = input_ids.shape
    vocab = id_map.shape[0]

    pad_v = -(-vocab // (NW * 128)) * (NW * 128)
    idm = jnp.pad(id_map.astype(jnp.int32), (0, pad_v - vocab))

    sg = _gather_rows_sc(table, idm, chunk=128)
    small = _layernorm_tc(
        sg, gamma.reshape(1, D), beta.reshape(1, D), blk=512
    )

    flat_ids = input_ids.reshape(-1).astype(jnp.int32)
    out = _gather_rows_sc(small, flat_ids, chunk=128)
    return out.reshape(batch, seq, D)


# speedup vs baseline: 8.2237x; 1.0947x over previous
import functools

import jax
import jax.numpy as jnp
from jax import lax
from jax.experimental import pallas as pl
from jax.experimental.pallas import tpu as pltpu
from jax.experimental.pallas import tpu_sc as plsc

D = 120
EPS = 1e-12
NC = 2
NS = 16
NW = NC * NS


def _gather_rows_sc(table, idx, chunk, nbuf, look):
    B = idx.shape[0]
    V, d = table.shape
    b_per_w = B // NW
    n_chunks = b_per_w // chunk
    assert b_per_w * NW == B and n_chunks * chunk == b_per_w
    assert n_chunks % nbuf == 0 and 0 < look < nbuf
    mesh = plsc.VectorSubcoreMesh(core_axis_name="c", subcore_axis_name="s")

    @functools.partial(
        pl.kernel,
        out_type=jax.ShapeDtypeStruct((B, d), table.dtype),
        mesh=mesh,
        compiler_params=pltpu.CompilerParams(use_tc_tiling_on_sc=False),
        scratch_types=[
            pltpu.VMEM((b_per_w,), jnp.int32),
            pltpu.VMEM((nbuf, chunk, d), table.dtype),
        ]
        + [pltpu.SemaphoreType.DMA] * (2 * nbuf),
    )
    def k(table_hbm, idx_hbm, out_hbm, idx_v, rows_v, *sems):
        gsem, ssem = sems[:nbuf], sems[nbuf:]
        wid = lax.axis_index("s") * NC + lax.axis_index("c")
        base = wid * b_per_w

        def gather(i, b, sliced):
            src = table_hbm.at[idx_v.at[pl.ds(i * chunk, chunk)]]
            return pltpu.make_async_copy(src, rows_v.at[b], gsem[b]) if sliced \
                else pltpu.async_copy(src, rows_v.at[b], gsem[b])

        def scatter(i, b):
            return pltpu.make_async_copy(
                rows_v.at[b], out_hbm.at[pl.ds(base + i * chunk, chunk)], ssem[b]
            )

        pltpu.sync_copy(idx_hbm.at[pl.ds(base, b_per_w)], idx_v)
        for j in range(look):
            gather(j, j % nbuf, False)

        def group(g, _):
            i0 = g * nbuf
            for b in range(nbuf):
                i = i0 + b
                j = i + look
                jb = (b + look) % nbuf

                @pl.when(j < n_chunks)
                def _():
                    @pl.when(j - nbuf >= 0)
                    def _():
                        scatter(j - nbuf, jb).wait()

                    gather(j, jb, False)

                gather(i, b, True).wait()
                scatter(i, b).start()
            return ()

        lax.fori_loop(0, n_chunks // nbuf, group, ())
        for b in range(nbuf):
            scatter(n_chunks - nbuf + b, b).wait()

    return k(table, idx)


def _layernorm_tc(x, gamma, beta, blk):
    rows, d = x.shape

    def body(x_ref, g_ref, b_ref, o_ref):
        v = x_ref[...]
        mean = jnp.mean(v, axis=-1, keepdims=True)
        cen = v - mean
        var = jnp.mean(cen * cen, axis=-1, keepdims=True)
        o_ref[...] = cen * lax.rsqrt(var + EPS) * g_ref[...] + b_ref[...]

    return pl.pallas_call(
        body,
        grid=(rows // blk,),
        in_specs=[
            pl.BlockSpec((blk, d), lambda i: (i, 0)),
            pl.BlockSpec((1, d), lambda i: (0, 0)),
            pl.BlockSpec((1, d), lambda i: (0, 0)),
        ],
        out_specs=pl.BlockSpec((blk, d), lambda i: (i, 0)),
        out_shape=jax.ShapeDtypeStruct((rows, d), jnp.float32),
    )(x, gamma, beta)


def kernel(input_ids, id_map, table, gamma, beta):
    batch, seq = input_ids.shape
    vocab = id_map.shape[0]

    pad_v = -(-vocab // (NW * 128)) * (NW * 128)
    idm = jnp.pad(id_map.astype(jnp.int32), (0, pad_v - vocab))

    sg = _gather_rows_sc(table, idm, chunk=128, nbuf=2, look=1)
    small = _layernorm_tc(
        sg, gamma.reshape(1, D), beta.reshape(1, D), blk=512
    )

    flat_ids = input_ids.reshape(-1).astype(jnp.int32)
    out = _gather_rows_sc(small, flat_ids, chunk=128, nbuf=4, look=2)
    return out.reshape(batch, seq, D)


# speedup vs baseline: 9.7285x; 1.1830x over previous
import functools

import jax
import jax.numpy as jnp
from jax import lax
from jax.experimental import pallas as pl
from jax.experimental.pallas import tpu as pltpu
from jax.experimental.pallas import tpu_sc as plsc

D = 120
DP = 128
EPS = 1e-12
NC = 2
NS = 16
NW = NC * NS
L = 16


def _layernorm_table_tc(table, gamma, beta, blk):
    rows, d = table.shape

    def body(x_ref, g_ref, b_ref, o_ref):
        v = x_ref[...]
        mean = jnp.mean(v, axis=-1, keepdims=True)
        cen = v - mean
        var = jnp.mean(cen * cen, axis=-1, keepdims=True)
        o_ref[...] = cen * lax.rsqrt(var + EPS) * g_ref[...] + b_ref[...]

    return pl.pallas_call(
        body,
        grid=(rows // blk,),
        in_specs=[
            pl.BlockSpec((blk, d), lambda i: (i, 0)),
            pl.BlockSpec((1, d), lambda i: (0, 0)),
            pl.BlockSpec((1, d), lambda i: (0, 0)),
        ],
        out_specs=pl.BlockSpec((blk, d), lambda i: (i, 0)),
        out_shape=jax.ShapeDtypeStruct((rows, d), jnp.float32),
    )(table, gamma, beta)


def _lookup_sc(normed, ids, id_map, chunk, nbuf, look):
    B = ids.shape[0]
    V = id_map.shape[0]
    b_per_w = B // NW
    n_chunks = b_per_w // chunk
    assert b_per_w * NW == B and n_chunks * chunk == b_per_w
    assert n_chunks % nbuf == 0 and 0 < look < nbuf
    mesh = plsc.VectorSubcoreMesh(core_axis_name="c", subcore_axis_name="s")

    @functools.partial(
        pl.kernel,
        out_type=jax.ShapeDtypeStruct((B, D), jnp.float32),
        mesh=mesh,
        compiler_params=pltpu.CompilerParams(
            use_tc_tiling_on_sc=False, needs_layout_passes=False
        ),
        scratch_types=[
            pltpu.VMEM((V,), jnp.int32),
            pltpu.VMEM((b_per_w,), jnp.int32),
            pltpu.VMEM((nbuf, chunk, D), jnp.float32),
            pltpu.SemaphoreType.DMA,
        ]
        + [pltpu.SemaphoreType.DMA] * (2 * nbuf),
    )
    def k(normed_hbm, ids_hbm, idm_hbm, out_hbm, idm_v, ids_v, rows_v, sem0, *sems):
        gsem, ssem = sems[:nbuf], sems[nbuf:]
        wid = lax.axis_index("s") * NC + lax.axis_index("c")
        base = wid * b_per_w

        cp = pltpu.async_copy(idm_hbm, idm_v, sem0)
        pltpu.sync_copy(ids_hbm.at[pl.ds(base, b_per_w)], ids_v)
        cp.wait()

        def conv(i, _):
            v = ids_v[pl.ds(i * L, L)]
            ids_v[pl.ds(i * L, L)] = plsc.load_gather(idm_v, [v])
            return ()

        lax.fori_loop(0, b_per_w // L, conv, ())

        def gather(i, b, make_only):
            src = normed_hbm.at[ids_v.at[pl.ds(i * chunk, chunk)]]
            return pltpu.make_async_copy(src, rows_v.at[b], gsem[b]) if make_only \
                else pltpu.async_copy(src, rows_v.at[b], gsem[b])

        def scatter(i, b):
            return pltpu.make_async_copy(
                rows_v.at[b],
                out_hbm.at[pl.ds(base + i * chunk, chunk)],
                ssem[b],
            )

        for j in range(look):
            gather(j, j % nbuf, False)

        def group(g, _):
            i0 = g * nbuf
            for b in range(nbuf):
                i = i0 + b
                j = i + look
                jb = (b + look) % nbuf

                @pl.when(j < n_chunks)
                def _():
                    @pl.when(j - nbuf >= 0)
                    def _():
                        scatter(j - nbuf, jb).wait()

                    gather(j, jb, False)

                gather(i, b, True).wait()
                scatter(i, b).start()
            return ()

        lax.fori_loop(0, n_chunks // nbuf, group, ())
        for b in range(nbuf):
            scatter(n_chunks - nbuf + b, b).wait()

    return k(normed, ids, id_map)


def kernel(input_ids, id_map, table, gamma, beta):
    batch, seq = input_ids.shape
    normed = _layernorm_table_tc(
        table, gamma.reshape(1, D), beta.reshape(1, D), blk=1000
    )
    flat_ids = input_ids.reshape(-1).astype(jnp.int32)
    out = _lookup_sc(
        normed, flat_ids, id_map.astype(jnp.int32), chunk=128, nbuf=4, look=2
    )
    return out.reshape(batch, seq, D)


# speedup vs baseline: 17.2759x; 1.7758x over previous
import functools

import jax
import jax.numpy as jnp
from jax import lax
from jax.experimental import pallas as pl
from jax.experimental.pallas import tpu as pltpu
from jax.experimental.pallas import tpu_sc as plsc

D = 120
DP = 128
EPS = 1e-12
NC = 2
NS = 16
NW = NC * NS
L = 16


def _layernorm_table_tc(table, gamma, beta, blk):
    rows, d = table.shape

    def body(x_ref, g_ref, b_ref, o_ref):
        v = x_ref[...]
        mean = jnp.mean(v, axis=-1, keepdims=True)
        cen = v - mean
        var = jnp.mean(cen * cen, axis=-1, keepdims=True)
        y = cen * lax.rsqrt(var + EPS) * g_ref[...] + b_ref[...]
        o_ref[...] = jnp.pad(y, ((0, 0), (0, DP - d)))

    return pl.pallas_call(
        body,
        grid=(rows // blk,),
        in_specs=[
            pl.BlockSpec((blk, d), lambda i: (i, 0)),
            pl.BlockSpec((1, d), lambda i: (0, 0)),
            pl.BlockSpec((1, d), lambda i: (0, 0)),
        ],
        out_specs=pl.BlockSpec((blk, DP), lambda i: (i, 0)),
        out_shape=jax.ShapeDtypeStruct((rows, DP), jnp.float32),
    )(table, gamma, beta)


def _lookup_sc(normed, ids, id_map, chunk, nbuf, look):
    B = ids.shape[0]
    V = id_map.shape[0]
    b_per_w = B // NW
    n_chunks = b_per_w // chunk
    assert b_per_w * NW == B and n_chunks * chunk == b_per_w
    assert n_chunks % nbuf == 0 and 0 < look < nbuf
    mesh = plsc.VectorSubcoreMesh(core_axis_name="c", subcore_axis_name="s")

    @functools.partial(
        pl.kernel,
        out_type=jax.ShapeDtypeStruct((B, DP), jnp.float32),
        mesh=mesh,
        compiler_params=pltpu.CompilerParams(
            use_tc_tiling_on_sc=True, needs_layout_passes=False
        ),
        scratch_types=[
            pltpu.VMEM((V,), jnp.int32),
            pltpu.VMEM((b_per_w,), jnp.int32),
            pltpu.VMEM((nbuf, chunk, DP), jnp.float32),
            pltpu.SemaphoreType.DMA,
        ]
        + [pltpu.SemaphoreType.DMA] * (2 * nbuf),
    )
    def k(normed_hbm, ids_hbm, idm_hbm, out_hbm, idm_v, ids_v, rows_v, sem0, *sems):
        gsem, ssem = sems[:nbuf], sems[nbuf:]
        wid = lax.axis_index("s") * NC + lax.axis_index("c")
        base = wid * b_per_w

        cp = pltpu.async_copy(idm_hbm, idm_v, sem0)
        pltpu.sync_copy(ids_hbm.at[pl.ds(base, b_per_w)], ids_v)
        cp.wait()

        def conv(i, _):
            v = ids_v[pl.ds(i * L, L)]
            ids_v[pl.ds(i * L, L)] = plsc.load_gather(idm_v, [v])
            return ()

        lax.fori_loop(0, b_per_w // L, conv, ())

        def gather(i, b, make_only):
            src = normed_hbm.at[ids_v.at[pl.ds(i * chunk, chunk)]]
            return pltpu.make_async_copy(src, rows_v.at[b], gsem[b]) if make_only \
                else pltpu.async_copy(src, rows_v.at[b], gsem[b])

        def scatter(i, b):
            return pltpu.make_async_copy(
                rows_v.at[b],
                out_hbm.at[pl.ds(base + i * chunk, chunk)],
                ssem[b],
            )

        for j in range(look):
            gather(j, j % nbuf, False)

        def group(g, _):
            i0 = g * nbuf
            for b in range(nbuf):
                i = i0 + b
                j = i + look
                jb = (b + look) % nbuf

                @pl.when(j < n_chunks)
                def _():
                    @pl.when(j - nbuf >= 0)
                    def _():
                        scatter(j - nbuf, jb).wait()

                    gather(j, jb, False)

                gather(i, b, True).wait()
                scatter(i, b).start()
            return ()

        lax.fori_loop(0, n_chunks // nbuf, group, ())
        for b in range(nbuf):
            scatter(n_chunks - nbuf + b, b).wait()

    return k(normed, ids, id_map)


def kernel(input_ids, id_map, table, gamma, beta):
    batch, seq = input_ids.shape
    normed = _layernorm_table_tc(
        table, gamma.reshape(1, D), beta.reshape(1, D), blk=1000
    )
    flat_ids = input_ids.reshape(-1).astype(jnp.int32)
    out = _lookup_sc(
        normed, flat_ids, id_map.astype(jnp.int32), chunk=128, nbuf=4, look=2
    )
    return out[:, :D].reshape(batch, seq, D)


# speedup vs baseline: 17.8249x; 1.0318x over previous
import functools

import jax
import jax.numpy as jnp
from jax import lax
from jax.experimental import pallas as pl
from jax.experimental.pallas import tpu as pltpu
from jax.experimental.pallas import tpu_sc as plsc

D = 120
DP = 128
EPS = 1e-12
NC = 2
NS = 16
NW = NC * NS
L = 16


def _layernorm_table_tc(table, gamma, beta, blk):
    rows, d = table.shape

    def body(x_ref, g_ref, b_ref, o_ref):
        v = x_ref[...]
        mean = jnp.mean(v, axis=-1, keepdims=True)
        cen = v - mean
        var = jnp.mean(cen * cen, axis=-1, keepdims=True)
        y = cen * lax.rsqrt(var + EPS) * g_ref[...] + b_ref[...]
        o_ref[...] = jnp.pad(y, ((0, 0), (0, DP - d)))

    return pl.pallas_call(
        body,
        grid=(rows // blk,),
        in_specs=[
            pl.BlockSpec((blk, d), lambda i: (i, 0)),
            pl.BlockSpec((1, d), lambda i: (0, 0)),
            pl.BlockSpec((1, d), lambda i: (0, 0)),
        ],
        out_specs=pl.BlockSpec((blk, DP), lambda i: (i, 0)),
        out_shape=jax.ShapeDtypeStruct((rows, DP), jnp.float32),
    )(table, gamma, beta)


def _lookup_sc(normed, ids, id_map, chunk, nbuf, look):
    B = ids.shape[0]
    V = id_map.shape[0]
    b_per_w = B // NW
    n_chunks = b_per_w // chunk
    assert b_per_w * NW == B and n_chunks * chunk == b_per_w
    assert n_chunks % nbuf == 0 and 0 < look < nbuf
    mesh = plsc.VectorSubcoreMesh(core_axis_name="c", subcore_axis_name="s")

    @functools.partial(
        pl.kernel,
        out_type=jax.ShapeDtypeStruct((B, DP), jnp.float32),
        mesh=mesh,
        compiler_params=pltpu.CompilerParams(
            use_tc_tiling_on_sc=True, needs_layout_passes=False
        ),
        scratch_types=[
            pltpu.VMEM((V,), jnp.int32),
            pltpu.VMEM((b_per_w,), jnp.int32),
            pltpu.VMEM((nbuf, chunk, DP), jnp.float32),
            pltpu.SemaphoreType.DMA,
        ]
        + [pltpu.SemaphoreType.DMA] * (2 * nbuf),
    )
    def k(normed_hbm, ids_hbm, idm_hbm, out_hbm, idm_v, ids_v, rows_v, sem0, *sems):
        gsem, ssem = sems[:nbuf], sems[nbuf:]
        wid = lax.axis_index("s") * NC + lax.axis_index("c")
        base = wid * b_per_w

        cp = pltpu.async_copy(idm_hbm, idm_v, sem0)
        pltpu.sync_copy(ids_hbm.at[pl.ds(base, b_per_w)], ids_v)
        cp.wait()

        def conv(i, _):
            v = ids_v[pl.ds(i * L, L)]
            ids_v[pl.ds(i * L, L)] = plsc.load_gather(idm_v, [v])
            return ()

        lax.fori_loop(0, b_per_w // L, conv, ())

        def gather(i, b, make_only):
            src = normed_hbm.at[ids_v.at[pl.ds(i * chunk, chunk)]]
            return pltpu.make_async_copy(src, rows_v.at[b], gsem[b]) if make_only \
                else pltpu.async_copy(src, rows_v.at[b], gsem[b])

        def scatter(i, b):
            return pltpu.make_async_copy(
                rows_v.at[b],
                out_hbm.at[pl.ds(base + i * chunk, chunk)],
                ssem[b],
            )

        for j in range(look):
            gather(j, j % nbuf, False)

        def group(g, _):
            i0 = g * nbuf
            for b in range(nbuf):
                i = i0 + b
                j = i + look
                jb = (b + look) % nbuf

                @pl.when(j < n_chunks)
                def _():
                    @pl.when(j - nbuf >= 0)
                    def _():
                        scatter(j - nbuf, jb).wait()

                    gather(j, jb, False)

                gather(i, b, True).wait()
                scatter(i, b).start()
            return ()

        lax.fori_loop(0, n_chunks // nbuf, group, ())
        for b in range(nbuf):
            scatter(n_chunks - nbuf + b, b).wait()

    return k(normed, ids, id_map)


def kernel(input_ids, id_map, table, gamma, beta):
    batch, seq = input_ids.shape
    normed = _layernorm_table_tc(
        table, gamma.reshape(1, D), beta.reshape(1, D), blk=2000
    )
    flat_ids = input_ids.reshape(-1).astype(jnp.int32)
    out = _lookup_sc(
        normed, flat_ids, id_map.astype(jnp.int32), chunk=128, nbuf=5, look=3
    )
    return out[:, :D].reshape(batch, seq, D)


# speedup vs baseline: 18.3400x; 1.0289x over previous
import functools

import jax
import jax.numpy as jnp
from jax import lax
from jax.experimental import pallas as pl
from jax.experimental.pallas import tpu as pltpu
from jax.experimental.pallas import tpu_sc as plsc

D = 120
DP = 128
EPS = 1e-12
NC = 2
NS = 16
NW = NC * NS
L = 16

_SC_PARAMS = pltpu.CompilerParams(
    use_tc_tiling_on_sc=True, needs_layout_passes=False
)
_MESH = dict(core_axis_name="c", subcore_axis_name="s")


def _translate_sc(ids2, id_map):
    batch, seq = ids2.shape
    V = id_map.shape[0]
    rows_per_w = batch // NW
    t_per_w = rows_per_w * seq
    assert rows_per_w * NW == batch and t_per_w % L == 0

    @functools.partial(
        pl.kernel,
        out_type=jax.ShapeDtypeStruct((batch * seq,), jnp.int32),
        mesh=plsc.VectorSubcoreMesh(**_MESH),
        compiler_params=_SC_PARAMS,
        scratch_types=[
            pltpu.VMEM((rows_per_w, seq), jnp.int32),
            pltpu.VMEM((V,), jnp.int32),
            pltpu.VMEM((t_per_w,), jnp.int32),
            pltpu.SemaphoreType.DMA,
        ],
    )
    def k(ids_hbm, idm_hbm, out_hbm, ids2_v, idm_v, cidx_v, sem):
        wid = lax.axis_index("s") * NC + lax.axis_index("c")
        cp = pltpu.async_copy(idm_hbm, idm_v, sem)
        pltpu.sync_copy(ids_hbm.at[pl.ds(wid * rows_per_w, rows_per_w)], ids2_v)
        cp.wait()

        def conv(g, _):
            t = g * L + lax.iota(jnp.int32, L)
            r = t // seq
            c = t - r * seq
            v = plsc.load_gather(ids2_v, [r, c])
            cidx_v[pl.ds(g * L, L)] = plsc.load_gather(idm_v, [v])
            return ()

        lax.fori_loop(0, t_per_w // L, conv, ())
        pltpu.sync_copy(cidx_v, out_hbm.at[pl.ds(wid * t_per_w, t_per_w)])

    return k(ids2, id_map)


def _layernorm_table_tc(table, gamma, beta, blk):
    rows, d = table.shape

    def body(x_ref, g_ref, b_ref, o_ref):
        v = x_ref[...]
        mean = jnp.mean(v, axis=-1, keepdims=True)
        cen = v - mean
        var = jnp.mean(cen * cen, axis=-1, keepdims=True)
        y = cen * lax.rsqrt(var + EPS) * g_ref[...] + b_ref[...]
        o_ref[...] = jnp.pad(y, ((0, 0), (0, DP - d)))

    return pl.pallas_call(
        body,
        grid=(rows // blk,),
        in_specs=[
            pl.BlockSpec((blk, d), lambda i: (i, 0)),
            pl.BlockSpec((1, d), lambda i: (0, 0)),
            pl.BlockSpec((1, d), lambda i: (0, 0)),
        ],
        out_specs=pl.BlockSpec((blk, DP), lambda i: (i, 0)),
        out_shape=jax.ShapeDtypeStruct((rows, DP), jnp.float32),
    )(table, gamma, beta)


def _gather_sc(normed, cidx, chunk, nbuf, look):
    B = cidx.shape[0]
    b_per_w = B // NW
    n_chunks = b_per_w // chunk
    assert b_per_w * NW == B and n_chunks * chunk == b_per_w
    assert n_chunks % nbuf == 0 and 0 < look < nbuf

    @functools.partial(
        pl.kernel,
        out_type=jax.ShapeDtypeStruct((B, DP), jnp.float32),
        mesh=plsc.VectorSubcoreMesh(**_MESH),
        compiler_params=_SC_PARAMS,
        scratch_types=[
            pltpu.VMEM((b_per_w,), jnp.int32),
            pltpu.VMEM((nbuf, chunk, DP), jnp.float32),
        ]
        + [pltpu.SemaphoreType.DMA] * (2 * nbuf),
    )
    def k(normed_hbm, cidx_hbm, out_hbm, cidx_v, rows_v, *sems):
        gsem, ssem = sems[:nbuf], sems[nbuf:]
        wid = lax.axis_index("s") * NC + lax.axis_index("c")
        base = wid * b_per_w
        pltpu.sync_copy(cidx_hbm.at[pl.ds(base, b_per_w)], cidx_v)

        def gather(i, b, make_only):
            src = normed_hbm.at[cidx_v.at[pl.ds(i * chunk, chunk)]]
            return pltpu.make_async_copy(src, rows_v.at[b], gsem[b]) if make_only \
                else pltpu.async_copy(src, rows_v.at[b], gsem[b])

        def scatter(i, b):
            return pltpu.make_async_copy(
                rows_v.at[b],
                out_hbm.at[pl.ds(base + i * chunk, chunk)],
                ssem[b],
            )

        for j in range(look):
            gather(j, j % nbuf, False)

        def group(g, _):
            i0 = g * nbuf
            for b in range(nbuf):
                i = i0 + b
                j = i + look
                jb = (b + look) % nbuf

                @pl.when(j < n_chunks)
                def _():
                    @pl.when(j - nbuf >= 0)
                    def _():
                        scatter(j - nbuf, jb).wait()

                    gather(j, jb, False)

                gather(i, b, True).wait()
                scatter(i, b).start()
            return ()

        lax.fori_loop(0, n_chunks // nbuf, group, ())
        for b in range(nbuf):
            scatter(n_chunks - nbuf + b, b).wait()

    return k(normed, cidx)


def kernel(input_ids, id_map, table, gamma, beta):
    batch, seq = input_ids.shape
    cidx = _translate_sc(input_ids.astype(jnp.int32), id_map.astype(jnp.int32))
    normed = _layernorm_table_tc(
        table, gamma.reshape(1, D), beta.reshape(1, D), blk=2000
    )
    out = _gather_sc(normed, cidx, chunk=128, nbuf=5, look=3)
    return out[:, :D].reshape(batch, seq, D)


# speedup vs baseline: 18.3456x; 1.0003x over previous
import functools

import jax
import jax.numpy as jnp
from jax import lax
from jax.experimental import pallas as pl
from jax.experimental.pallas import tpu as pltpu
from jax.experimental.pallas import tpu_sc as plsc

D = 120
DP = 128
EPS = 1e-12
NC = 2
NS = 16
NW = NC * NS
L = 16

_SC_PARAMS = pltpu.CompilerParams(
    use_tc_tiling_on_sc=True, needs_layout_passes=False
)
_MESH = dict(core_axis_name="c", subcore_axis_name="s")


def _translate_sc(ids2, id_map):
    batch, seq = ids2.shape
    V = id_map.shape[0]
    rows_per_w = batch // NW
    t_per_w = rows_per_w * seq
    assert rows_per_w * NW == batch and t_per_w % L == 0

    @functools.partial(
        pl.kernel,
        out_type=jax.ShapeDtypeStruct((batch * seq,), jnp.int32),
        mesh=plsc.VectorSubcoreMesh(**_MESH),
        compiler_params=_SC_PARAMS,
        scratch_types=[
            pltpu.VMEM((rows_per_w, seq), jnp.int32),
            pltpu.VMEM((V,), jnp.int32),
            pltpu.VMEM((t_per_w,), jnp.int32),
            pltpu.SemaphoreType.DMA,
        ],
    )
    def k(ids_hbm, idm_hbm, out_hbm, ids2_v, idm_v, cidx_v, sem):
        wid = lax.axis_index("s") * NC + lax.axis_index("c")
        cp = pltpu.async_copy(idm_hbm, idm_v, sem)
        pltpu.sync_copy(ids_hbm.at[pl.ds(wid * rows_per_w, rows_per_w)], ids2_v)
        cp.wait()

        def conv(g, _):
            t = g * L + lax.iota(jnp.int32, L)
            r = t // seq
            c = t - r * seq
            v = plsc.load_gather(ids2_v, [r, c])
            cidx_v[pl.ds(g * L, L)] = plsc.load_gather(idm_v, [v])
            return ()

        lax.fori_loop(0, t_per_w // L, conv, ())
        pltpu.sync_copy(cidx_v, out_hbm.at[pl.ds(wid * t_per_w, t_per_w)])

    return k(ids2, id_map)


def _layernorm_table_tc(table, gamma, beta, blk):
    rows, d = table.shape

    def body(x_ref, g_ref, b_ref, o_ref):
        v = x_ref[...]
        mean = jnp.mean(v, axis=-1, keepdims=True)
        cen = v - mean
        var = jnp.mean(cen * cen, axis=-1, keepdims=True)
        y = cen * lax.rsqrt(var + EPS) * g_ref[...] + b_ref[...]
        o_ref[...] = jnp.pad(y, ((0, 0), (0, DP - d)))

    return pl.pallas_call(
        body,
        grid=(rows // blk,),
        in_specs=[
            pl.BlockSpec((blk, d), lambda i: (i, 0)),
            pl.BlockSpec((1, d), lambda i: (0, 0)),
            pl.BlockSpec((1, d), lambda i: (0, 0)),
        ],
        out_specs=pl.BlockSpec((blk, DP), lambda i: (i, 0)),
        out_shape=jax.ShapeDtypeStruct((rows, DP), jnp.float32),
    )(table, gamma, beta)


def _gather_sc(normed, cidx, chunk, nbuf, look):
    B = cidx.shape[0]
    b_per_w = B // NW
    n_chunks = b_per_w // chunk
    assert b_per_w * NW == B and n_chunks * chunk == b_per_w
    assert n_chunks % nbuf == 0 and 0 < look < nbuf

    @functools.partial(
        pl.kernel,
        out_type=jax.ShapeDtypeStruct((B, DP), jnp.float32),
        mesh=plsc.VectorSubcoreMesh(**_MESH),
        compiler_params=_SC_PARAMS,
        scratch_types=[
            pltpu.VMEM((b_per_w,), jnp.int32),
            pltpu.VMEM((nbuf, chunk, DP), jnp.float32),
        ]
        + [pltpu.SemaphoreType.DMA] * (2 * nbuf),
    )
    def k(normed_hbm, cidx_hbm, out_hbm, cidx_v, rows_v, *sems):
        gsem, ssem = sems[:nbuf], sems[nbuf:]
        wid = lax.axis_index("s") * NC + lax.axis_index("c")
        base = wid * b_per_w
        pltpu.sync_copy(cidx_hbm.at[pl.ds(base, b_per_w)], cidx_v)

        def gather(i, b, make_only):
            src = normed_hbm.at[cidx_v.at[pl.ds(i * chunk, chunk)]]
            return pltpu.make_async_copy(src, rows_v.at[b], gsem[b]) if make_only \
                else pltpu.async_copy(src, rows_v.at[b], gsem[b])

        def scatter(i, b):
            return pltpu.make_async_copy(
                rows_v.at[b],
                out_hbm.at[pl.ds(base + i * chunk, chunk)],
                ssem[b],
            )

        for j in range(look):
            gather(j, j % nbuf, False)

        def group(g, _):
            i0 = g * nbuf
            for b in range(nbuf):
                i = i0 + b
                j = i + look
                jb = (b + look) % nbuf

                @pl.when(j < n_chunks)
                def _():
                    @pl.when(j - nbuf >= 0)
                    def _():
                        scatter(j - nbuf, jb).wait()

                    gather(j, jb, False)

                gather(i, b, True).wait()
                scatter(i, b).start()
            return ()

        lax.fori_loop(0, n_chunks // nbuf, group, ())
        for b in range(nbuf):
            scatter(n_chunks - nbuf + b, b).wait()

    return k(normed, cidx)


def kernel(input_ids, id_map, table, gamma, beta):
    batch, seq = input_ids.shape
    cidx = _translate_sc(input_ids.astype(jnp.int32), id_map.astype(jnp.int32))
    normed = _layernorm_table_tc(
        table, gamma.reshape(1, D), beta.reshape(1, D), blk=2000
    )
    out = _gather_sc(normed, cidx, chunk=128, nbuf=5, look=4)
    return out[:, :D].reshape(batch, seq, D)
